# Initial kernel scaffold; baseline (speedup 1.0000x reference)
import jax
import jax.numpy as jnp
from jax.experimental import pallas as pl
from jax.experimental.pallas import tpu as pltpu

H, B, N, D = 3, 4, 2048, 2048
K = N // 2
HB = H * B
BN = 256
NBLK = N // BN
SIGN = -2 ** 31


def _body(x_ref, w_ref, src_ref, out_ref, score_ref):
    hb = pl.program_id(0)
    jn = pl.program_id(1)

    x = x_ref[0]
    w = w_ref[...]
    s = jax.lax.dot_general(x, w, (((1,), (1,)), ((), ())),
                            preferred_element_type=jnp.float32)
    score_ref[pl.ds(hb, 1), pl.ds(jn * BN, BN)] = s.reshape(1, BN)

    @pl.when(jnp.logical_and(hb == HB - 1, jn == NBLK - 1))
    def _():
        s_all = score_ref[...]
        s_all = jnp.where(s_all == 0.0, 0.0, s_all)
        bits = jax.lax.bitcast_convert_type(s_all, jnp.int32)
        skey = jnp.where(bits >= 0, bits, bits ^ jnp.int32(0x7FFFFFFF))

        sign = jnp.int32(SIGN)
        acc = jnp.zeros((HB, 1), jnp.int32)
        for i in range(31, -1, -1):
            bit = sign if i == 31 else jnp.int32(1 << i)
            t = acc | bit
            c = jnp.sum((skey < (t ^ sign)).astype(jnp.int32),
                        axis=-1, keepdims=True)
            acc = jnp.where(c < K, t, acc)
        vs = acc ^ sign
        c_lt = jnp.sum((skey < vs).astype(jnp.int32), axis=-1, keepdims=True)
        r = K - c_lt

        eq = skey == vs
        idx = jax.lax.broadcasted_iota(jnp.int32, (HB, N), 1)
        acc2 = jnp.zeros((HB, 1), jnp.int32)
        for i in range(10, -1, -1):
            t = acc2 | jnp.int32(1 << i)
            c = jnp.sum((eq & (idx < t)).astype(jnp.int32),
                        axis=-1, keepdims=True)
            acc2 = jnp.where(c < r, t, acc2)
        sel = (skey < vs) | (eq & (idx <= acc2))
        sel_f = sel.astype(jnp.float32)

        rows = jax.lax.broadcasted_iota(jnp.int32, (B, HB), 0)
        cols = jax.lax.broadcasted_iota(jnp.int32, (B, HB), 1)
        gather_b = (cols % B == rows).astype(jnp.float32)
        cnt = jax.lax.dot_general(gather_b, sel_f, (((1,), (0,)), ((), ())),
                                  preferred_element_type=jnp.float32)
        src = src_ref[...]
        out_ref[...] = cnt + (float(H) - cnt) * src


def kernel(x_list, src_mask, W, b):
    x = x_list.reshape(HB, N, D)
    src = src_mask.reshape(B, N)
    mask = pl.pallas_call(
        _body,
        grid=(HB, NBLK),
        in_specs=[
            pl.BlockSpec((1, BN, D), lambda i, j: (i, j, 0)),
            pl.BlockSpec((1, D), lambda i, j: (0, 0)),
            pl.BlockSpec((B, N), lambda i, j: (0, 0)),
        ],
        out_specs=pl.BlockSpec((B, N), lambda i, j: (0, 0)),
        out_shape=jax.ShapeDtypeStruct((B, N), jnp.float32),
        scratch_shapes=[pltpu.VMEM((HB, N), jnp.float32)],
        compiler_params=pltpu.CompilerParams(
            dimension_semantics=("arbitrary", "arbitrary")),
    )(x, W, src)
    return x_list, mask.reshape(B, 1, N)


# speedup vs baseline: 1.0936x; 1.0936x over previous
import jax
import jax.numpy as jnp
from jax.experimental import pallas as pl
from jax.experimental.pallas import tpu as pltpu

H, B, N, D = 3, 4, 2048, 2048
K = N // 2
HB = H * B
BN = 256
NBLK = N // BN
SIGN = -2 ** 31


def _body(x_ref, w_ref, src_ref, out_ref, score_ref):
    hb = pl.program_id(0)
    jn = pl.program_id(1)

    x = x_ref[0].astype(jnp.bfloat16).astype(jnp.float32)
    w = w_ref[...].astype(jnp.bfloat16).astype(jnp.float32)
    s = jax.lax.dot_general(x, w, (((1,), (1,)), ((), ())),
                            preferred_element_type=jnp.float32)
    score_ref[pl.ds(hb, 1), pl.ds(jn * BN, BN)] = s.reshape(1, BN)

    @pl.when(jnp.logical_and(hb == HB - 1, jn == NBLK - 1))
    def _():
        s_all = score_ref[...]
        s_all = jnp.where(s_all == 0.0, 0.0, s_all)
        bits = jax.lax.bitcast_convert_type(s_all, jnp.int32)
        skey = jnp.where(bits >= 0, bits, bits ^ jnp.int32(0x7FFFFFFF))

        sign = jnp.int32(SIGN)
        acc = jnp.zeros((HB, 1), jnp.int32)
        for i in range(31, -1, -1):
            bit = sign if i == 31 else jnp.int32(1 << i)
            t = acc | bit
            c = jnp.sum((skey < (t ^ sign)).astype(jnp.int32),
                        axis=-1, keepdims=True)
            acc = jnp.where(c < K, t, acc)
        vs = acc ^ sign
        c_lt = jnp.sum((skey < vs).astype(jnp.int32), axis=-1, keepdims=True)
        r = K - c_lt

        eq = skey == vs
        idx = jax.lax.broadcasted_iota(jnp.int32, (HB, N), 1)
        acc2 = jnp.zeros((HB, 1), jnp.int32)
        for i in range(10, -1, -1):
            t = acc2 | jnp.int32(1 << i)
            c = jnp.sum((eq & (idx < t)).astype(jnp.int32),
                        axis=-1, keepdims=True)
            acc2 = jnp.where(c < r, t, acc2)
        sel = (skey < vs) | (eq & (idx <= acc2))
        sel_f = sel.astype(jnp.float32)

        rows = jax.lax.broadcasted_iota(jnp.int32, (B, HB), 0)
        cols = jax.lax.broadcasted_iota(jnp.int32, (B, HB), 1)
        gather_b = (cols % B == rows).astype(jnp.float32)
        cnt = jax.lax.dot_general(gather_b, sel_f, (((1,), (0,)), ((), ())),
                                  preferred_element_type=jnp.float32)
        src = src_ref[...]
        out_ref[...] = cnt + (float(H) - cnt) * src


def kernel(x_list, src_mask, W, b):
    x = x_list.reshape(HB, N, D)
    src = src_mask.reshape(B, N)
    mask = pl.pallas_call(
        _body,
        grid=(HB, NBLK),
        in_specs=[
            pl.BlockSpec((1, BN, D), lambda i, j: (i, j, 0)),
            pl.BlockSpec((1, D), lambda i, j: (0, 0)),
            pl.BlockSpec((B, N), lambda i, j: (0, 0)),
        ],
        out_specs=pl.BlockSpec((B, N), lambda i, j: (0, 0)),
        out_shape=jax.ShapeDtypeStruct((B, N), jnp.float32),
        scratch_shapes=[pltpu.VMEM((HB, N), jnp.float32)],
        compiler_params=pltpu.CompilerParams(
            dimension_semantics=("arbitrary", "arbitrary")),
    )(x, W, src)
    return x_list, mask.reshape(B, 1, N)


# speedup vs baseline: 1.6372x; 1.4970x over previous
import jax
import jax.numpy as jnp
from jax.experimental import pallas as pl
from jax.experimental.pallas import tpu as pltpu

H, B, N, D = 3, 4, 2048, 2048
K = N // 2
HB = H * B
BN = 256
NBLK = N // BN
SIGN = -2 ** 31


def _body(x_ref, w_ref, src_ref, xout_ref, out_ref, score_ref):
    hb = pl.program_id(0)
    jn = pl.program_id(1)

    xraw = x_ref[...]
    xout_ref[...] = xraw

    x = xraw[0].astype(jnp.bfloat16).astype(jnp.float32)
    w = w_ref[...].astype(jnp.bfloat16).astype(jnp.float32)
    s = jax.lax.dot_general(x, w, (((1,), (1,)), ((), ())),
                            preferred_element_type=jnp.float32)
    score_ref[pl.ds(hb, 1), pl.ds(jn * BN, BN)] = s.reshape(1, BN)

    @pl.when(jnp.logical_and(hb == HB - 1, jn == NBLK - 1))
    def _():
        s_all = score_ref[...]
        s_all = jnp.where(s_all == 0.0, 0.0, s_all)
        bits = jax.lax.bitcast_convert_type(s_all, jnp.int32)
        skey = jnp.where(bits >= 0, bits, bits ^ jnp.int32(0x7FFFFFFF))

        sign = jnp.int32(SIGN)
        acc = jnp.zeros((HB, 1), jnp.int32)
        for i in range(31, -1, -1):
            bit = sign if i == 31 else jnp.int32(1 << i)
            t = acc | bit
            c = jnp.sum((skey < (t ^ sign)).astype(jnp.int32),
                        axis=-1, keepdims=True)
            acc = jnp.where(c < K, t, acc)
        vs = acc ^ sign
        c_lt = jnp.sum((skey < vs).astype(jnp.int32), axis=-1, keepdims=True)
        r = K - c_lt

        eq = skey == vs
        idx = jax.lax.broadcasted_iota(jnp.int32, (HB, N), 1)
        acc2 = jnp.zeros((HB, 1), jnp.int32)
        for i in range(10, -1, -1):
            t = acc2 | jnp.int32(1 << i)
            c = jnp.sum((eq & (idx < t)).astype(jnp.int32),
                        axis=-1, keepdims=True)
            acc2 = jnp.where(c < r, t, acc2)
        sel = (skey < vs) | (eq & (idx <= acc2))
        sel_f = sel.astype(jnp.float32)

        rows = jax.lax.broadcasted_iota(jnp.int32, (B, HB), 0)
        cols = jax.lax.broadcasted_iota(jnp.int32, (B, HB), 1)
        gather_b = (cols % B == rows).astype(jnp.float32)
        cnt = jax.lax.dot_general(gather_b, sel_f, (((1,), (0,)), ((), ())),
                                  preferred_element_type=jnp.float32)
        src = src_ref[...]
        out_ref[...] = cnt + (float(H) - cnt) * src


def kernel(x_list, src_mask, W, b):
    x = x_list.reshape(HB, N, D)
    src = src_mask.reshape(B, N)
    x_out, mask = pl.pallas_call(
        _body,
        grid=(HB, NBLK),
        in_specs=[
            pl.BlockSpec((1, BN, D), lambda i, j: (i, j, 0)),
            pl.BlockSpec((1, D), lambda i, j: (0, 0)),
            pl.BlockSpec((B, N), lambda i, j: (0, 0)),
        ],
        out_specs=[
            pl.BlockSpec((1, BN, D), lambda i, j: (i, j, 0)),
            pl.BlockSpec((B, N), lambda i, j: (0, 0)),
        ],
        out_shape=[
            jax.ShapeDtypeStruct((HB, N, D), jnp.float32),
            jax.ShapeDtypeStruct((B, N), jnp.float32),
        ],
        scratch_shapes=[pltpu.VMEM((HB, N), jnp.float32)],
        compiler_params=pltpu.CompilerParams(
            dimension_semantics=("arbitrary", "arbitrary")),
    )(x, W, src)
    return x_out.reshape(H, B, N, D), mask.reshape(B, 1, N)
